# Initial kernel scaffold; baseline (speedup 1.0000x reference)
import jax
import jax.numpy as jnp
from jax.experimental import pallas as pl


def kernel(node_ids, messages, node_memories, W_conv, W_lin, b_lin, ln_gamma, ln_beta):
    raise NotImplementedError("write your pallas kernel here")


# speedup vs baseline: 1.3718x; 1.3718x over previous
import functools

import jax
import jax.numpy as jnp
from jax import lax
from jax.experimental import pallas as pl
from jax.experimental.pallas import tpu as pltpu
from jax.experimental.pallas import tpu_sc as plsc

NUM_NODES = 100000
MEM_DIM = 128
MSG_DIM = 128
PERIOD = 7
B = 16384
D_IN = MSG_DIM + MEM_DIM

NC = 2
NS = 16
L = 16
NW = NC * NS
BPW = B // NW

PAD_ROWS = 8
TOT = NUM_NODES + PAD_ROWS

CP_WORKERS = NW - 1
CP_CHUNK = 200
CP_STEPS = 17
CP_ROWS = CP_CHUNK * CP_STEPS

_mesh = plsc.VectorSubcoreMesh(
    core_axis_name="c", subcore_axis_name="s", num_cores=NC, num_subcores=NS)


def _worker_id():
  return lax.axis_index("s") * NC + lax.axis_index("c")


@functools.partial(
    pl.kernel,
    out_type=jax.ShapeDtypeStruct((B, MEM_DIM), jnp.float32),
    mesh=_mesh,
    compiler_params=pltpu.CompilerParams(needs_layout_passes=False),
    scratch_types=[
        pltpu.VMEM((BPW,), jnp.int32),
        pltpu.VMEM((BPW, MEM_DIM), jnp.float32),
        pltpu.SemaphoreType.DMA,
    ],
)
def _gather_rows(table_hbm, idx_hbm, out_hbm, idx_v, rows_v, sem):
  base = _worker_id() * BPW
  pltpu.sync_copy(idx_hbm.at[pl.ds(base, BPW)], idx_v)
  pltpu.async_copy(table_hbm.at[idx_v], rows_v, sem).wait()
  pltpu.sync_copy(rows_v, out_hbm.at[pl.ds(base, BPW)])


def _compute_body(msg_ref, old_ref, wm_ref, wl_ref, bl_ref, g_ref, bt_ref,
                  out_ref):
  x1 = msg_ref[...] * wm_ref[:, :MSG_DIM]
  x2 = old_ref[...] * wm_ref[:, MSG_DIM:]
  dn = (((1,), (1,)), ((), ()))
  acc = lax.dot_general(x1, wl_ref[:, :MSG_DIM], dn,
                        preferred_element_type=jnp.float32,
                        precision=lax.Precision.HIGHEST)
  acc = acc + lax.dot_general(x2, wl_ref[:, MSG_DIM:], dn,
                              preferred_element_type=jnp.float32,
                              precision=lax.Precision.HIGHEST)
  acc = acc + bl_ref[...]
  mean = jnp.mean(acc, axis=-1, keepdims=True)
  var = jnp.mean((acc - mean) ** 2, axis=-1, keepdims=True)
  out_ref[...] = (acc - mean) / jnp.sqrt(var + 1e-5) * g_ref[...] + bt_ref[...]


_BLK = 1024
_compute = pl.pallas_call(
    _compute_body,
    grid=(B // _BLK,),
    in_specs=[
        pl.BlockSpec((_BLK, MSG_DIM), lambda i: (i, 0)),
        pl.BlockSpec((_BLK, MEM_DIM), lambda i: (i, 0)),
        pl.BlockSpec((1, D_IN), lambda i: (0, 0)),
        pl.BlockSpec((MEM_DIM, D_IN), lambda i: (0, 0)),
        pl.BlockSpec((1, MEM_DIM), lambda i: (0, 0)),
        pl.BlockSpec((1, MEM_DIM), lambda i: (0, 0)),
        pl.BlockSpec((1, MEM_DIM), lambda i: (0, 0)),
    ],
    out_specs=pl.BlockSpec((_BLK, MEM_DIM), lambda i: (i, 0)),
    out_shape=jax.ShapeDtypeStruct((B, MEM_DIM), jnp.float32),
)


@functools.partial(
    pl.kernel,
    out_type=(
        jax.ShapeDtypeStruct((TOT, MEM_DIM), jnp.float32),
        jax.ShapeDtypeStruct((B,), jnp.int32),
    ),
    mesh=_mesh,
    compiler_params=pltpu.CompilerParams(needs_layout_passes=False),
    scratch_types=[
        pltpu.VMEM((NUM_NODES,), jnp.int32),
        pltpu.VMEM((BPW,), jnp.int32),
        pltpu.VMEM((BPW,), jnp.int32),
        pltpu.VMEM((CP_CHUNK, MEM_DIM), jnp.float32),
    ],
)
def _copy_winner(table_hbm, idx_hbm, out_hbm, mask_hbm,
                 winner_v, idsb, maskb, cbuf):
  wid = _worker_id()
  iota = lax.iota(jnp.int32, L)

  @pl.when(wid == 0)
  def _build():

    def build_chunk(ci):
      pltpu.sync_copy(idx_hbm.at[pl.ds(ci * BPW, BPW)], idsb)
      for j in range(BPW // L):
        idv = idsb[pl.ds(j * L, L)]
        bv = ci * BPW + j * L + iota
        _, is_last = plsc.scan_count(idv)
        plsc.store_scatter(winner_v, [idv], bv, mask=is_last)

    lax.fori_loop(0, B // BPW, lambda ci, _: (build_chunk(ci), 0)[1], 0)

    def mask_chunk(ci):
      pltpu.sync_copy(idx_hbm.at[pl.ds(ci * BPW, BPW)], idsb)
      for j in range(BPW // L):
        idv = idsb[pl.ds(j * L, L)]
        bv = ci * BPW + j * L + iota
        w = plsc.load_gather(winner_v, [idv])
        maskb[pl.ds(j * L, L)] = (w == bv).astype(jnp.int32)
      pltpu.sync_copy(maskb, mask_hbm.at[pl.ds(ci * BPW, BPW)])

    lax.fori_loop(0, B // BPW, lambda ci, _: (mask_chunk(ci), 0)[1], 0)

  @pl.when(wid > 0)
  def _copy():
    start = (wid - 1) * CP_ROWS

    def cp(k):
      off = jnp.minimum(start + k * CP_CHUNK, NUM_NODES - CP_CHUNK)
      pltpu.sync_copy(table_hbm.at[pl.ds(off, CP_CHUNK)], cbuf)
      pltpu.sync_copy(cbuf, out_hbm.at[pl.ds(off, CP_CHUNK)])

    lax.fori_loop(0, CP_STEPS, lambda k, _: (cp(k), 0)[1], 0)


_SUB = 128

@functools.partial(
    pl.kernel,
    out_type=(),
    mesh=_mesh,
    compiler_params=pltpu.CompilerParams(needs_layout_passes=False),
    scratch_types=[
        pltpu.VMEM((BPW,), jnp.int32),
        pltpu.VMEM((BPW,), jnp.int32),
        pltpu.VMEM((BPW // _SUB, _SUB), jnp.int32),
        pltpu.VMEM((_SUB, MEM_DIM), jnp.float32),
        pltpu.SemaphoreType.DMA,
    ],
)
def _scatter_rows(idx_hbm, mask_hbm, rows_hbm, out_hbm,
                  idsb, maskb, sidx, rbuf, sem):
  base = _worker_id() * BPW
  iota = lax.iota(jnp.int32, L)
  pltpu.sync_copy(idx_hbm.at[pl.ds(base, BPW)], idsb)
  pltpu.sync_copy(mask_hbm.at[pl.ds(base, BPW)], maskb)
  for sc in range(BPW // _SUB):
    for j in range(_SUB // L):
      o = sc * _SUB + j * L
      idv = idsb[pl.ds(o, L)]
      mv = maskb[pl.ds(o, L)]
      red = jnp.where(mv == 1, idv, NUM_NODES + (iota & (PAD_ROWS - 1)))
      sidx[sc, pl.ds(j * L, L)] = red
    pltpu.sync_copy(rows_hbm.at[pl.ds(base + sc * _SUB, _SUB)], rbuf)
    pltpu.async_copy(rbuf, out_hbm.at[sidx.at[sc]], sem).wait()


def kernel(node_ids, messages, node_memories, W_conv, W_lin, b_lin,
           ln_gamma, ln_beta):
  ids = node_ids.astype(jnp.int32)
  w_mid = W_conv[:, 0, PERIOD // 2].reshape(1, D_IN)
  old = _gather_rows(node_memories, ids)
  normed = _compute(messages, old, w_mid, W_lin,
                    b_lin.reshape(1, MEM_DIM), ln_gamma.reshape(1, MEM_DIM),
                    ln_beta.reshape(1, MEM_DIM))
  out0, mask = _copy_winner(node_memories, ids)
  tbl = jax.new_ref(out0)
  _scatter_rows(ids, mask, normed, tbl)
  return tbl[...][:NUM_NODES]
